# baseline (speedup 1.0000x reference)
import functools
import math

import jax
import jax.numpy as jnp
from jax import lax
from jax.experimental import pallas as pl
from jax.experimental.pallas import tpu as pltpu

_NODE_TYPES = ['SB', 'PV', 'PQ', 'NB']
_EDGE_TYPES = [('SB', 'PV'), ('SB', 'PQ'), ('SB', 'NB'), ('PV', 'PQ'),
               ('PV', 'NB'), ('PV', 'PV'), ('PQ', 'NB'), ('PQ', 'PQ'),
               ('NB', 'NB')]
_N = 2500
_E = 35000
_D = 128
_B = 500
_NBLK = _E // _B
_NT = len(_EDGE_TYPES)
_F32 = jnp.float32


def _conv_body(src_ref, dst_ref, ea_ref, xsrc_ref, xdst_ref,
               wq_ref, bq_ref, wk_ref, bk_ref, wv_ref, bv_ref,
               we_ref, be_ref, wskip_ref, bskip_ref,
               out_ref, q_s, k_s, v_s, acc_s, den_s):
    i = pl.program_id(1)

    @pl.when(i == 0)
    def _init():
        xs = xsrc_ref[0]
        xd = xdst_ref[0]
        q_s[...] = jnp.dot(xd, wq_ref[0], preferred_element_type=_F32) + bq_ref[0]
        k_s[...] = jnp.dot(xs, wk_ref[0], preferred_element_type=_F32) + bk_ref[0]
        v_s[...] = jnp.dot(xs, wv_ref[0], preferred_element_type=_F32) + bv_ref[0]
        acc_s[...] = jnp.zeros_like(acc_s)
        den_s[...] = jnp.zeros_like(den_s)

    src = src_ref[0]
    dst = dst_ref[0]
    iota = lax.broadcasted_iota(jnp.int32, (_N, _B), 0)
    oh_s = (iota == jnp.broadcast_to(src, (_N, _B))).astype(_F32)
    oh_d = (iota == jnp.broadcast_to(dst, (_N, _B))).astype(_F32)

    dn_t = (((0,), (0,)), ((), ()))
    dn_s = (((1,), (0,)), ((), ()))
    bf = jnp.bfloat16
    oh_s16 = oh_s.astype(bf)
    oh_d16 = oh_d.astype(bf)
    kb = lax.dot_general(oh_s16, k_s[...].astype(bf), dn_t, preferred_element_type=_F32)
    vb = lax.dot_general(oh_s16, v_s[...].astype(bf), dn_t, preferred_element_type=_F32)
    qb = lax.dot_general(oh_d16, q_s[...].astype(bf), dn_t, preferred_element_type=_F32)
    eb = jnp.dot(ea_ref[0], we_ref[0], preferred_element_type=_F32) + be_ref[0]

    inv_sqrt_d = 1.0 / math.sqrt(_D)
    logits = jnp.sum(qb * (kb + eb), axis=1, keepdims=True) * inv_sqrt_d
    ex = jnp.exp(logits)
    contrib = ex * (vb + eb)
    acc_s[...] += lax.dot_general(oh_d16, contrib.astype(bf), dn_s,
                                  preferred_element_type=_F32)
    den_s[...] += lax.dot_general(oh_d16, jnp.broadcast_to(ex, (_B, 8)).astype(bf),
                                  dn_s, preferred_element_type=_F32)

    @pl.when(i == _NBLK - 1)
    def _fin():
        skip = jnp.dot(xdst_ref[0], wskip_ref[0],
                       preferred_element_type=_F32) + bskip_ref[0]
        out_ref[0] = acc_s[...] / (den_s[:, 0:1] + 1e-16) + skip


@jax.jit
def _hetero_layer(src_all, dst_all, ea_all, xsrc_all, xdst_all,
                  wq, bq, wk, bk, wv, bv, we, be, wsk, bsk):
    full_nd = pl.BlockSpec((1, _N, _D), lambda t, i: (t, 0, 0))
    full_dd = pl.BlockSpec((1, _D, _D), lambda t, i: (t, 0, 0))
    full_b = pl.BlockSpec((1, 1, _D), lambda t, i: (t, 0, 0))
    idx_spec = pl.BlockSpec((1, 1, _B), lambda t, i: (t * _NBLK + i, 0, 0))
    ea_spec = pl.BlockSpec((1, _B, 2), lambda t, i: (t * _NBLK + i, 0, 0))
    return pl.pallas_call(
        _conv_body,
        grid=(_NT, _NBLK),
        in_specs=[idx_spec, idx_spec, ea_spec, full_nd, full_nd,
                  full_dd, full_b, full_dd, full_b, full_dd, full_b,
                  pl.BlockSpec((1, 2, _D), lambda t, i: (t, 0, 0)), full_b,
                  full_dd, full_b],
        out_specs=pl.BlockSpec((1, _N, _D), lambda t, i: (t, 0, 0)),
        out_shape=jax.ShapeDtypeStruct((_NT, _N, _D), _F32),
        scratch_shapes=[pltpu.VMEM((_N, _D), _F32)] * 4 + [pltpu.VMEM((_N, 8), _F32)],
        compiler_params=pltpu.CompilerParams(
            dimension_semantics=("arbitrary", "arbitrary")),
    )(src_all, dst_all, ea_all, xsrc_all, xdst_all,
      wq, bq, wk, bk, wv, bv, we, be, wsk, bsk)


def _linear_body(x_ref, w_ref, b_ref, o_ref):
    o_ref[...] = jnp.dot(x_ref[...], w_ref[...],
                         preferred_element_type=_F32) + b_ref[...]


@jax.jit
def _final_linear(x, w, b):
    return pl.pallas_call(
        _linear_body,
        out_shape=jax.ShapeDtypeStruct((_N, _D), _F32),
    )(x, w, b.reshape(1, _D))


def kernel(x_SB, x_PV, x_PQ, x_NB,
           edge_index_SB_PV, edge_index_SB_PQ, edge_index_SB_NB,
           edge_index_PV_PQ, edge_index_PV_NB, edge_index_PV_PV,
           edge_index_PQ_NB, edge_index_PQ_PQ, edge_index_NB_NB,
           edge_attr_SB_PV, edge_attr_SB_PQ, edge_attr_SB_NB,
           edge_attr_PV_PQ, edge_attr_PV_NB, edge_attr_PV_PV,
           edge_attr_PQ_NB, edge_attr_PQ_PQ, edge_attr_NB_NB,
           params):
    loc = dict(locals())
    eis = {'%s_%s' % (s, d): loc['edge_index_%s_%s' % (s, d)]
           for (s, d) in _EDGE_TYPES}
    eas = {'%s_%s' % (s, d): loc['edge_attr_%s_%s' % (s, d)]
           for (s, d) in _EDGE_TYPES}

    src_all = jnp.concatenate(
        [eis['%s_%s' % (s, d)][0].reshape(_NBLK, 1, _B)
         for (s, d) in _EDGE_TYPES], axis=0)
    dst_all = jnp.concatenate(
        [eis['%s_%s' % (s, d)][1].reshape(_NBLK, 1, _B)
         for (s, d) in _EDGE_TYPES], axis=0)
    ea_all = jnp.concatenate(
        [eas['%s_%s' % (s, d)].reshape(_NBLK, _B, 2)
         for (s, d) in _EDGE_TYPES], axis=0)

    x = {'SB': x_SB, 'PV': x_PV, 'PQ': x_PQ, 'NB': x_NB}
    for layer in params['convs']:
        xsrc_all = jnp.stack([x[s] for (s, d) in _EDGE_TYPES])
        xdst_all = jnp.stack([x[d] for (s, d) in _EDGE_TYPES])
        pk = ['%s_%s' % (s, d) for (s, d) in _EDGE_TYPES]
        wq = jnp.stack([layer[k]['Wq'] for k in pk])
        bq = jnp.stack([layer[k]['bq'] for k in pk]).reshape(_NT, 1, _D)
        wk = jnp.stack([layer[k]['Wk'] for k in pk])
        bk = jnp.stack([layer[k]['bk'] for k in pk]).reshape(_NT, 1, _D)
        wv = jnp.stack([layer[k]['Wv'] for k in pk])
        bv = jnp.stack([layer[k]['bv'] for k in pk]).reshape(_NT, 1, _D)
        we = jnp.stack([layer[k]['We'] for k in pk])
        be = jnp.stack([layer[k]['be'] for k in pk]).reshape(_NT, 1, _D)
        wsk = jnp.stack([layer[k]['Wskip'] for k in pk])
        bsk = jnp.stack([layer[k]['bskip'] for k in pk]).reshape(_NT, 1, _D)

        conv = _hetero_layer(src_all, dst_all, ea_all, xsrc_all, xdst_all,
                             wq, bq, wk, bk, wv, bv, we, be, wsk, bsk)

        agg = {}
        for ti, (s, d) in enumerate(_EDGE_TYPES):
            agg[d] = agg[d] + conv[ti] if d in agg else conv[ti]
        x = {nt: (jax.nn.relu(agg[nt]) if nt in agg else x[nt])
             for nt in _NODE_TYPES}

    return _final_linear(x['NB'], params['lin_w'], params['lin_b'])


# speedup vs baseline: 1.0495x; 1.0495x over previous
import functools
import math

import jax
import jax.numpy as jnp
from jax import lax
from jax.experimental import pallas as pl
from jax.experimental.pallas import tpu as pltpu
from jax.experimental.pallas import tpu_sc as plsc

_NODE_TYPES = ['SB', 'PV', 'PQ', 'NB']
_EDGE_TYPES = [('SB', 'PV'), ('SB', 'PQ'), ('SB', 'NB'), ('PV', 'PQ'),
               ('PV', 'NB'), ('PV', 'PV'), ('PQ', 'NB'), ('PQ', 'PQ'),
               ('NB', 'NB')]
_N = 2500
_NP = 2560
_E = 35000
_EP = 35328
_D = 128
_NT = 9
_NC = 2
_NS = 16
_CHUNK = _EP // (_NC * _NS)
_NG = _CHUNK // 16
_F32 = jnp.float32


def _qkvs_body(xs_ref, xd_ref, wq_ref, bq_ref, wk_ref, bk_ref, wv_ref,
               bv_ref, wsk_ref, bsk_ref, q_ref, kv_ref, sk_ref):
    xs = xs_ref[0]
    xd = xd_ref[0]
    q_ref[0] = jnp.dot(xd, wq_ref[0], preferred_element_type=_F32) + bq_ref[0]
    kv_ref[0, :, :_D] = jnp.dot(xs, wk_ref[0], preferred_element_type=_F32) + bk_ref[0]
    kv_ref[0, :, _D:] = jnp.dot(xs, wv_ref[0], preferred_element_type=_F32) + bv_ref[0]
    sk_ref[0] = jnp.dot(xd, wsk_ref[0], preferred_element_type=_F32) + bsk_ref[0]


@jax.jit
def _qkvs(xsrc_all, xdst_all, wq, bq, wk, bk, wv, bv, wsk, bsk):
    nd = pl.BlockSpec((1, _N, _D), lambda t: (t, 0, 0))
    dd = pl.BlockSpec((1, _D, _D), lambda t: (t, 0, 0))
    bb = pl.BlockSpec((1, 1, _D), lambda t: (t, 0, 0))
    return pl.pallas_call(
        _qkvs_body,
        grid=(_NT,),
        in_specs=[nd, nd, dd, bb, dd, bb, dd, bb, dd, bb],
        out_specs=[nd, pl.BlockSpec((1, _N, 2 * _D), lambda t: (t, 0, 0)), nd],
        out_shape=[jax.ShapeDtypeStruct((_NT, _N, _D), _F32),
                   jax.ShapeDtypeStruct((_NT, _N, 2 * _D), _F32),
                   jax.ShapeDtypeStruct((_NT, _N, _D), _F32)],
    )(xsrc_all, xdst_all, wq, bq, wk, bk, wv, bv, wsk, bsk)


_EB = 4416

def _emb_body(ea_ref, we_ref, be_ref, e_ref):
    e_ref[...] = jnp.dot(ea_ref[...], we_ref[0],
                         preferred_element_type=_F32) + be_ref[0]


@jax.jit
def _emb(ea_pad_all, we, be):
    nblk = (_NT * _EP) // _EB
    return pl.pallas_call(
        _emb_body,
        grid=(nblk,),
        in_specs=[pl.BlockSpec((_EB, 2), lambda i: (i, 0)),
                  pl.BlockSpec((1, 2, _D), lambda i: (i // 8, 0, 0)),
                  pl.BlockSpec((1, 1, _D), lambda i: (i // 8, 0, 0))],
        out_specs=pl.BlockSpec((_EB, _D), lambda i: (i, 0)),
        out_shape=jax.ShapeDtypeStruct((_NT * _EP, _D), _F32),
    )(ea_pad_all, we, be)


def _sc_body(kv_hbm, q_hbm, e_hbm, srcoff_hbm, dstoff_hbm, dstloc_hbm,
             w_hbm, z_acc_hbm, z_den_hbm,
             acc_hbm, den_hbm,
             idx_s, idx_d, idx_l, w_v, kv_v, q_v, e_v, con_v, den_v,
             lg_v, ex_v, acc_sh, den_sh, sem):
    c = lax.axis_index("c")
    s = lax.axis_index("s")
    inv_sqrt_d = 1.0 / math.sqrt(_D)
    for j in range(16):
        den_v[j] = jnp.zeros((16,), _F32)

    def per_type(t, _):
        pltpu.sync_copy(z_acc_hbm.at[pl.ds(s * 160, 160)],
                        acc_sh.at[pl.ds(s * 160, 160)])
        pltpu.sync_copy(z_den_hbm.at[pl.ds(s * 160, 160)],
                        den_sh.at[pl.ds(s * 160, 160)])
        plsc.subcore_barrier()

        base = t * _EP + (s * _NC + c) * _CHUNK

        def per_group(g, _):
            off = base + g * 16
            pltpu.sync_copy(srcoff_hbm.at[pl.ds(off, 16)], idx_s)
            pltpu.sync_copy(dstoff_hbm.at[pl.ds(off, 16)], idx_d)
            pltpu.sync_copy(dstloc_hbm.at[pl.ds(off, 16)], idx_l)
            pltpu.sync_copy(w_hbm.at[pl.ds(off, 16)], w_v)
            pltpu.async_copy(kv_hbm.at[idx_s], kv_v, sem).wait()
            pltpu.async_copy(q_hbm.at[idx_d], q_v, sem).wait()
            pltpu.sync_copy(e_hbm.at[pl.ds(off, 16)], e_v)

            lane = lax.iota(jnp.int32, 16)

            lg = jnp.zeros((16,), _F32)
            for j in range(16):
                acc = jnp.zeros((16,), _F32)
                for d8 in range(8):
                    sl = pl.ds(d8 * 16, 16)
                    acc += q_v[j, sl] * (kv_v[j, sl] + e_v[j, sl])
                vals = [acc[i] for i in range(16)]
                while len(vals) > 1:
                    vals = [vals[i] + vals[i + 1]
                            for i in range(0, len(vals), 2)]
                lg = jnp.where(lane == j, jnp.full((16,), vals[0]), lg)
            ex = jnp.exp(lg * inv_sqrt_d) * w_v[...]
            for j in range(16):
                exj = jnp.full((16,), ex[j])
                for d8 in range(8):
                    sl = pl.ds(d8 * 16, 16)
                    con_v[j, sl] = (kv_v[j, pl.ds(_D + d8 * 16, 16)]
                                    + e_v[j, sl]) * exj
                den_v[j] = exj
            pltpu.sync_copy(con_v, acc_sh.at[idx_l], add=True)
            pltpu.sync_copy(den_v, den_sh.at[idx_l], add=True)
            return _

        lax.fori_loop(0, _NG, per_group, 0)
        plsc.subcore_barrier()
        pltpu.sync_copy(acc_sh.at[pl.ds(s * 160, 160)],
                        acc_hbm.at[c, t, pl.ds(s * 160, 160)])
        pltpu.sync_copy(den_sh.at[pl.ds(s * 160, 160)],
                        den_hbm.at[c, t, pl.ds(s * 160, 160)])
        plsc.subcore_barrier()
        return _

    lax.fori_loop(0, _NT, per_type, 0)


@jax.jit
def _sc_edge_phase(kv_flat, q_flat, e_flat, srcoff, dstoff, dstloc, w,
                   z_acc, z_den):
    mesh = plsc.VectorSubcoreMesh(core_axis_name="c", subcore_axis_name="s")
    f = functools.partial(
        pl.kernel,
        mesh=mesh,
        out_type=[jax.ShapeDtypeStruct((_NC, _NT, _NP, _D), _F32),
                  jax.ShapeDtypeStruct((_NC, _NT, _NP, 16), _F32)],
        scratch_types=[
            pltpu.VMEM((16,), jnp.int32),
            pltpu.VMEM((16,), jnp.int32),
            pltpu.VMEM((16,), jnp.int32),
            pltpu.VMEM((16,), _F32),
            pltpu.VMEM((16, 2 * _D), _F32),
            pltpu.VMEM((16, _D), _F32),
            pltpu.VMEM((16, _D), _F32),
            pltpu.VMEM((16, _D), _F32),
            pltpu.VMEM((16, 16), _F32),
            pltpu.VMEM((16,), _F32),
            pltpu.VMEM((16,), _F32),
            pltpu.VMEM_SHARED((_NP, _D), _F32),
            pltpu.VMEM_SHARED((_NP, 16), _F32),
            pltpu.SemaphoreType.DMA,
        ],
    )(_sc_body)
    return f(kv_flat, q_flat, e_flat, srcoff, dstoff, dstloc, w, z_acc, z_den)


def _norm_body(a0_ref, a1_ref, d0_ref, d1_ref, sk_ref, o_ref):
    den = d0_ref[0, 0][:, 0:1] + d1_ref[0, 0][:, 0:1] + 1e-16
    o_ref[0] = (a0_ref[0, 0] + a1_ref[0, 0]) / den + sk_ref[0]


@jax.jit
def _normalize(acc, den, skip_pad):
    a_spec0 = pl.BlockSpec((1, 1, _NP, _D), lambda t: (0, t, 0, 0))
    a_spec1 = pl.BlockSpec((1, 1, _NP, _D), lambda t: (1, t, 0, 0))
    d_spec0 = pl.BlockSpec((1, 1, _NP, 16), lambda t: (0, t, 0, 0))
    d_spec1 = pl.BlockSpec((1, 1, _NP, 16), lambda t: (1, t, 0, 0))
    return pl.pallas_call(
        _norm_body,
        grid=(_NT,),
        in_specs=[a_spec0, a_spec1, d_spec0, d_spec1,
                  pl.BlockSpec((1, _NP, _D), lambda t: (t, 0, 0))],
        out_specs=pl.BlockSpec((1, _NP, _D), lambda t: (t, 0, 0)),
        out_shape=jax.ShapeDtypeStruct((_NT, _NP, _D), _F32),
    )(acc, acc, den, den, skip_pad)


def _linear_body(x_ref, w_ref, b_ref, o_ref):
    o_ref[...] = jnp.dot(x_ref[...], w_ref[...],
                         preferred_element_type=_F32) + b_ref[...]


@jax.jit
def _final_linear(x, w, b):
    return pl.pallas_call(
        _linear_body,
        out_shape=jax.ShapeDtypeStruct((_N, _D), _F32),
    )(x, w, b.reshape(1, _D))


def kernel(x_SB, x_PV, x_PQ, x_NB,
           edge_index_SB_PV, edge_index_SB_PQ, edge_index_SB_NB,
           edge_index_PV_PQ, edge_index_PV_NB, edge_index_PV_PV,
           edge_index_PQ_NB, edge_index_PQ_PQ, edge_index_NB_NB,
           edge_attr_SB_PV, edge_attr_SB_PQ, edge_attr_SB_NB,
           edge_attr_PV_PQ, edge_attr_PV_NB, edge_attr_PV_PV,
           edge_attr_PQ_NB, edge_attr_PQ_PQ, edge_attr_NB_NB,
           params):
    loc = dict(locals())
    eis = {'%s_%s' % (s, d): loc['edge_index_%s_%s' % (s, d)]
           for (s, d) in _EDGE_TYPES}
    eas = {'%s_%s' % (s, d): loc['edge_attr_%s_%s' % (s, d)]
           for (s, d) in _EDGE_TYPES}

    pad = _EP - _E
    src_l, dst_l, ea_l, w_l = [], [], [], []
    for ti, (s, d) in enumerate(_EDGE_TYPES):
        ei = eis['%s_%s' % (s, d)]
        src_l.append(jnp.pad(ei[0], (0, pad)) + ti * _N)
        dst_l.append(jnp.pad(ei[1], (0, pad)))
        ea_l.append(jnp.pad(eas['%s_%s' % (s, d)], ((0, pad), (0, 0))))
        w_l.append(jnp.pad(jnp.ones((_E,), _F32), (0, pad)))
    srcoff = jnp.concatenate(src_l)
    dstloc = jnp.concatenate(dst_l)
    dstoff = jnp.concatenate(
        [dst_l[t] + t * _N for t in range(_NT)])
    ea_pad_all = jnp.concatenate(ea_l, axis=0)
    w = jnp.concatenate(w_l)
    z_acc = jnp.zeros((_NP, _D), _F32)
    z_den = jnp.zeros((_NP, 16), _F32)

    pk = ['%s_%s' % (s, d) for (s, d) in _EDGE_TYPES]
    x = {'SB': x_SB, 'PV': x_PV, 'PQ': x_PQ, 'NB': x_NB}
    for layer in params['convs']:
        xsrc_all = jnp.stack([x[s] for (s, d) in _EDGE_TYPES])
        xdst_all = jnp.stack([x[d] for (s, d) in _EDGE_TYPES])
        wq = jnp.stack([layer[k]['Wq'] for k in pk])
        bq = jnp.stack([layer[k]['bq'] for k in pk]).reshape(_NT, 1, _D)
        wk = jnp.stack([layer[k]['Wk'] for k in pk])
        bk = jnp.stack([layer[k]['bk'] for k in pk]).reshape(_NT, 1, _D)
        wv = jnp.stack([layer[k]['Wv'] for k in pk])
        bv = jnp.stack([layer[k]['bv'] for k in pk]).reshape(_NT, 1, _D)
        we = jnp.stack([layer[k]['We'] for k in pk])
        be = jnp.stack([layer[k]['be'] for k in pk]).reshape(_NT, 1, _D)
        wsk = jnp.stack([layer[k]['Wskip'] for k in pk])
        bsk = jnp.stack([layer[k]['bskip'] for k in pk]).reshape(_NT, 1, _D)

        q, kv, sk = _qkvs(xsrc_all, xdst_all, wq, bq, wk, bk, wv, bv,
                          wsk, bsk)
        e_flat = _emb(ea_pad_all, we, be)
        acc, den = _sc_edge_phase(kv.reshape(_NT * _N, 2 * _D),
                                  q.reshape(_NT * _N, _D),
                                  e_flat, srcoff, dstoff, dstloc, w,
                                  z_acc, z_den)
        sk_pad = jnp.pad(sk, ((0, 0), (0, _NP - _N), (0, 0)))
        conv = _normalize(acc, den, sk_pad)[:, :_N, :]

        agg = {}
        for ti, (s, d) in enumerate(_EDGE_TYPES):
            agg[d] = agg[d] + conv[ti] if d in agg else conv[ti]
        x = {nt: (jax.nn.relu(agg[nt]) if nt in agg else x[nt])
             for nt in _NODE_TYPES}

    return _final_linear(x['NB'], params['lin_w'], params['lin_b'])


# speedup vs baseline: 1.4443x; 1.3762x over previous
import functools
import math

import jax
import jax.numpy as jnp
from jax import lax
from jax.experimental import pallas as pl
from jax.experimental.pallas import tpu as pltpu
from jax.experimental.pallas import tpu_sc as plsc

_NODE_TYPES = ['SB', 'PV', 'PQ', 'NB']
_EDGE_TYPES = [('SB', 'PV'), ('SB', 'PQ'), ('SB', 'NB'), ('PV', 'PQ'),
               ('PV', 'NB'), ('PV', 'PV'), ('PQ', 'NB'), ('PQ', 'PQ'),
               ('NB', 'NB')]
_N = 2500
_NP = 2560
_E = 35000
_EP = 35840
_D = 128
_NT = 9
_NC = 2
_NS = 16
_G = 32
_CHUNK = _EP // (_NC * _NS)
_NG = _CHUNK // _G
_F32 = jnp.float32


def _qkvs_body(xs_ref, xd_ref, wq_ref, bq_ref, wk_ref, bk_ref, wv_ref,
               bv_ref, wsk_ref, bsk_ref, q_ref, kv_ref, sk_ref):
    xs = xs_ref[0]
    xd = xd_ref[0]
    q_ref[0] = jnp.dot(xd, wq_ref[0], preferred_element_type=_F32) + bq_ref[0]
    kv_ref[0, :, :_D] = jnp.dot(xs, wk_ref[0], preferred_element_type=_F32) + bk_ref[0]
    kv_ref[0, :, _D:] = jnp.dot(xs, wv_ref[0], preferred_element_type=_F32) + bv_ref[0]
    sk_ref[0] = jnp.dot(xd, wsk_ref[0], preferred_element_type=_F32) + bsk_ref[0]


@jax.jit
def _qkvs(xsrc_all, xdst_all, wq, bq, wk, bk, wv, bv, wsk, bsk):
    nd = pl.BlockSpec((1, _N, _D), lambda t: (t, 0, 0))
    dd = pl.BlockSpec((1, _D, _D), lambda t: (t, 0, 0))
    bb = pl.BlockSpec((1, 1, _D), lambda t: (t, 0, 0))
    return pl.pallas_call(
        _qkvs_body,
        grid=(_NT,),
        in_specs=[nd, nd, dd, bb, dd, bb, dd, bb, dd, bb],
        out_specs=[nd, pl.BlockSpec((1, _N, 2 * _D), lambda t: (t, 0, 0)), nd],
        out_shape=[jax.ShapeDtypeStruct((_NT, _N, _D), _F32),
                   jax.ShapeDtypeStruct((_NT, _N, 2 * _D), _F32),
                   jax.ShapeDtypeStruct((_NT, _N, _D), _F32)],
    )(xsrc_all, xdst_all, wq, bq, wk, bk, wv, bv, wsk, bsk)


_EB = 4480

def _emb_body(ea_ref, we_ref, be_ref, e_ref):
    e_ref[...] = jnp.dot(ea_ref[...], we_ref[0],
                         preferred_element_type=_F32) + be_ref[0]


@jax.jit
def _emb(ea_pad_all, we, be):
    nblk = (_NT * _EP) // _EB
    return pl.pallas_call(
        _emb_body,
        grid=(nblk,),
        in_specs=[pl.BlockSpec((_EB, 2), lambda i: (i, 0)),
                  pl.BlockSpec((1, 2, _D), lambda i: (i // 8, 0, 0)),
                  pl.BlockSpec((1, 1, _D), lambda i: (i // 8, 0, 0))],
        out_specs=pl.BlockSpec((_EB, _D), lambda i: (i, 0)),
        out_shape=jax.ShapeDtypeStruct((_NT * _EP, _D), _F32),
    )(ea_pad_all, we, be)


def _sc_body(kv_hbm, q_hbm, e_hbm, srcoff_hbm, dstoff_hbm, dstloc_hbm,
             w_hbm, z_acc_hbm, z_den_hbm,
             acc_hbm, den_hbm,
             idx_s, idx_d, idx_l, w_v, kv_v, q_v, e_v, con_v, den_v,
             lg_v, ex_v, acc_sh, den_sh, sem):
    c = lax.axis_index("c")
    s = lax.axis_index("s")
    inv_sqrt_d = 1.0 / math.sqrt(_D)
    for j in range(_G):
        den_v[j] = jnp.zeros((16,), _F32)

    def per_type(t, _):
        pltpu.sync_copy(z_acc_hbm.at[pl.ds(s * 160, 160)],
                        acc_sh.at[pl.ds(s * 160, 160)])
        pltpu.sync_copy(z_den_hbm.at[pl.ds(s * 160, 160)],
                        den_sh.at[pl.ds(s * 160, 160)])
        plsc.subcore_barrier()

        base = t * _EP + (s * _NC + c) * _CHUNK

        def per_group(g, _):
            off = base + g * _G
            pltpu.sync_copy(srcoff_hbm.at[pl.ds(off, _G)], idx_s)
            pltpu.sync_copy(dstoff_hbm.at[pl.ds(off, _G)], idx_d)
            pltpu.sync_copy(dstloc_hbm.at[pl.ds(off, _G)], idx_l)
            pltpu.sync_copy(w_hbm.at[pl.ds(off, _G)], w_v)
            cp1 = pltpu.async_copy(kv_hbm.at[idx_s], kv_v, sem)
            cp2 = pltpu.async_copy(q_hbm.at[idx_d], q_v, sem)
            cp3 = pltpu.async_copy(e_hbm.at[pl.ds(off, _G)], e_v, sem)
            cp1.wait()
            cp2.wait()
            cp3.wait()

            lane = lax.iota(jnp.int32, 16)
            for h in range(2):
                lg = jnp.zeros((16,), _F32)
                for jj in range(16):
                    j = h * 16 + jj
                    acc = jnp.zeros((16,), _F32)
                    for d8 in range(8):
                        sl = pl.ds(d8 * 16, 16)
                        acc += q_v[j, sl] * (kv_v[j, sl] + e_v[j, sl])
                    vals = [acc[i] for i in range(16)]
                    while len(vals) > 1:
                        vals = [vals[i] + vals[i + 1]
                                for i in range(0, len(vals), 2)]
                    lg = jnp.where(lane == jj, jnp.full((16,), vals[0]), lg)
                ex = jnp.exp(lg * inv_sqrt_d) * w_v[pl.ds(h * 16, 16)]
                for jj in range(16):
                    j = h * 16 + jj
                    exj = jnp.full((16,), ex[jj])
                    for d8 in range(8):
                        sl = pl.ds(d8 * 16, 16)
                        con_v[j, sl] = (kv_v[j, pl.ds(_D + d8 * 16, 16)]
                                        + e_v[j, sl]) * exj
                    den_v[j] = exj
            pltpu.sync_copy(con_v, acc_sh.at[idx_l], add=True)
            pltpu.sync_copy(den_v, den_sh.at[idx_l], add=True)
            return _

        lax.fori_loop(0, _NG, per_group, 0)
        plsc.subcore_barrier()
        pltpu.sync_copy(acc_sh.at[pl.ds(s * 160, 160)],
                        acc_hbm.at[c, t, pl.ds(s * 160, 160)])
        pltpu.sync_copy(den_sh.at[pl.ds(s * 160, 160)],
                        den_hbm.at[c, t, pl.ds(s * 160, 160)])
        plsc.subcore_barrier()
        return _

    lax.fori_loop(0, _NT, per_type, 0)


@jax.jit
def _sc_edge_phase(kv_flat, q_flat, e_flat, srcoff, dstoff, dstloc, w,
                   z_acc, z_den):
    mesh = plsc.VectorSubcoreMesh(core_axis_name="c", subcore_axis_name="s")
    f = functools.partial(
        pl.kernel,
        mesh=mesh,
        out_type=[jax.ShapeDtypeStruct((_NC, _NT, _NP, _D), _F32),
                  jax.ShapeDtypeStruct((_NC, _NT, _NP, 16), _F32)],
        scratch_types=[
            pltpu.VMEM((_G,), jnp.int32),
            pltpu.VMEM((_G,), jnp.int32),
            pltpu.VMEM((_G,), jnp.int32),
            pltpu.VMEM((_G,), _F32),
            pltpu.VMEM((_G, 2 * _D), _F32),
            pltpu.VMEM((_G, _D), _F32),
            pltpu.VMEM((_G, _D), _F32),
            pltpu.VMEM((_G, _D), _F32),
            pltpu.VMEM((_G, 16), _F32),
            pltpu.VMEM((16,), _F32),
            pltpu.VMEM((16,), _F32),
            pltpu.VMEM_SHARED((_NP, _D), _F32),
            pltpu.VMEM_SHARED((_NP, 16), _F32),
            pltpu.SemaphoreType.DMA,
        ],
    )(_sc_body)
    return f(kv_flat, q_flat, e_flat, srcoff, dstoff, dstloc, w, z_acc, z_den)


def _norm_body(a0_ref, a1_ref, d0_ref, d1_ref, sk_ref, o_ref):
    den = d0_ref[0, 0][:, 0:1] + d1_ref[0, 0][:, 0:1] + 1e-16
    o_ref[0] = (a0_ref[0, 0] + a1_ref[0, 0]) / den + sk_ref[0]


@jax.jit
def _normalize(acc, den, skip_pad):
    a_spec0 = pl.BlockSpec((1, 1, _NP, _D), lambda t: (0, t, 0, 0))
    a_spec1 = pl.BlockSpec((1, 1, _NP, _D), lambda t: (1, t, 0, 0))
    d_spec0 = pl.BlockSpec((1, 1, _NP, 16), lambda t: (0, t, 0, 0))
    d_spec1 = pl.BlockSpec((1, 1, _NP, 16), lambda t: (1, t, 0, 0))
    return pl.pallas_call(
        _norm_body,
        grid=(_NT,),
        in_specs=[a_spec0, a_spec1, d_spec0, d_spec1,
                  pl.BlockSpec((1, _NP, _D), lambda t: (t, 0, 0))],
        out_specs=pl.BlockSpec((1, _NP, _D), lambda t: (t, 0, 0)),
        out_shape=jax.ShapeDtypeStruct((_NT, _NP, _D), _F32),
    )(acc, acc, den, den, skip_pad)


def _linear_body(x_ref, w_ref, b_ref, o_ref):
    o_ref[...] = jnp.dot(x_ref[...], w_ref[...],
                         preferred_element_type=_F32) + b_ref[...]


@jax.jit
def _final_linear(x, w, b):
    return pl.pallas_call(
        _linear_body,
        out_shape=jax.ShapeDtypeStruct((_N, _D), _F32),
    )(x, w, b.reshape(1, _D))


def kernel(x_SB, x_PV, x_PQ, x_NB,
           edge_index_SB_PV, edge_index_SB_PQ, edge_index_SB_NB,
           edge_index_PV_PQ, edge_index_PV_NB, edge_index_PV_PV,
           edge_index_PQ_NB, edge_index_PQ_PQ, edge_index_NB_NB,
           edge_attr_SB_PV, edge_attr_SB_PQ, edge_attr_SB_NB,
           edge_attr_PV_PQ, edge_attr_PV_NB, edge_attr_PV_PV,
           edge_attr_PQ_NB, edge_attr_PQ_PQ, edge_attr_NB_NB,
           params):
    loc = dict(locals())
    eis = {'%s_%s' % (s, d): loc['edge_index_%s_%s' % (s, d)]
           for (s, d) in _EDGE_TYPES}
    eas = {'%s_%s' % (s, d): loc['edge_attr_%s_%s' % (s, d)]
           for (s, d) in _EDGE_TYPES}

    pad = _EP - _E
    src_l, dst_l, ea_l, w_l = [], [], [], []
    for ti, (s, d) in enumerate(_EDGE_TYPES):
        ei = eis['%s_%s' % (s, d)]
        src_l.append(jnp.pad(ei[0], (0, pad)) + ti * _N)
        dst_l.append(jnp.pad(ei[1], (0, pad)))
        ea_l.append(jnp.pad(eas['%s_%s' % (s, d)], ((0, pad), (0, 0))))
        w_l.append(jnp.pad(jnp.ones((_E,), _F32), (0, pad)))
    srcoff = jnp.concatenate(src_l)
    dstloc = jnp.concatenate(dst_l)
    dstoff = jnp.concatenate(
        [dst_l[t] + t * _N for t in range(_NT)])
    ea_pad_all = jnp.concatenate(ea_l, axis=0)
    w = jnp.concatenate(w_l)
    z_acc = jnp.zeros((_NP, _D), _F32)
    z_den = jnp.zeros((_NP, 16), _F32)

    pk = ['%s_%s' % (s, d) for (s, d) in _EDGE_TYPES]
    x = {'SB': x_SB, 'PV': x_PV, 'PQ': x_PQ, 'NB': x_NB}
    for layer in params['convs']:
        xsrc_all = jnp.stack([x[s] for (s, d) in _EDGE_TYPES])
        xdst_all = jnp.stack([x[d] for (s, d) in _EDGE_TYPES])
        wq = jnp.stack([layer[k]['Wq'] for k in pk])
        bq = jnp.stack([layer[k]['bq'] for k in pk]).reshape(_NT, 1, _D)
        wk = jnp.stack([layer[k]['Wk'] for k in pk])
        bk = jnp.stack([layer[k]['bk'] for k in pk]).reshape(_NT, 1, _D)
        wv = jnp.stack([layer[k]['Wv'] for k in pk])
        bv = jnp.stack([layer[k]['bv'] for k in pk]).reshape(_NT, 1, _D)
        we = jnp.stack([layer[k]['We'] for k in pk])
        be = jnp.stack([layer[k]['be'] for k in pk]).reshape(_NT, 1, _D)
        wsk = jnp.stack([layer[k]['Wskip'] for k in pk])
        bsk = jnp.stack([layer[k]['bskip'] for k in pk]).reshape(_NT, 1, _D)

        q, kv, sk = _qkvs(xsrc_all, xdst_all, wq, bq, wk, bk, wv, bv,
                          wsk, bsk)
        e_flat = _emb(ea_pad_all, we, be)
        acc, den = _sc_edge_phase(kv.reshape(_NT * _N, 2 * _D),
                                  q.reshape(_NT * _N, _D),
                                  e_flat, srcoff, dstoff, dstloc, w,
                                  z_acc, z_den)
        sk_pad = jnp.pad(sk, ((0, 0), (0, _NP - _N), (0, 0)))
        conv = _normalize(acc, den, sk_pad)[:, :_N, :]

        agg = {}
        for ti, (s, d) in enumerate(_EDGE_TYPES):
            agg[d] = agg[d] + conv[ti] if d in agg else conv[ti]
        x = {nt: (jax.nn.relu(agg[nt]) if nt in agg else x[nt])
             for nt in _NODE_TYPES}

    return _final_linear(x['NB'], params['lin_w'], params['lin_b'])
